# baseline (speedup 1.0000x reference)
import functools

import jax
import jax.numpy as jnp
from jax import lax
from jax.experimental import pallas as pl
from jax.experimental.pallas import tpu as pltpu
from jax.experimental.pallas import tpu_sc as plsc


def _fused_gather(table, il, ir):
    n_rows = il.shape[0] * 128
    D = table.shape[1]
    info = plsc.get_sparse_core_info()
    NC, NS = info.num_cores, info.num_subcores
    NW = NC * NS
    rows_per_w = n_rows // NW
    CHUNK = 256
    n_chunk = rows_per_w // CHUNK
    JPC = CHUNK // 128
    idx_rows_w = rows_per_w // 128

    mesh = plsc.VectorSubcoreMesh(core_axis_name="c", subcore_axis_name="s")

    @functools.partial(
        pl.kernel,
        mesh=mesh,
        compiler_params=pltpu.CompilerParams(use_tc_tiling_on_sc=False),
        out_type=jax.ShapeDtypeStruct((n_rows, 2 * D), jnp.float32),
        scratch_types=[
            pltpu.VMEM((idx_rows_w, 128), jnp.int32),
            pltpu.VMEM((idx_rows_w, 128), jnp.int32),
            pltpu.VMEM((2, CHUNK, D), jnp.float32),
            pltpu.VMEM((2, CHUNK, D), jnp.float32),
            pltpu.SemaphoreType.DMA,
            pltpu.SemaphoreType.DMA,
            pltpu.SemaphoreType.DMA,
            pltpu.SemaphoreType.DMA,
        ],
    )
    def gather_kernel(t_hbm, il_hbm, ir_hbm, o_hbm,
                      il_v, ir_v, rl_v, rr_v, sem_l, sem_r, sem_w0, sem_w1):
        wid = lax.axis_index("s") * NC + lax.axis_index("c")
        pltpu.sync_copy(il_hbm.at[pl.ds(wid * idx_rows_w, idx_rows_w)], il_v)
        pltpu.sync_copy(ir_hbm.at[pl.ds(wid * idx_rows_w, idx_rows_w)], ir_v)
        wsems = (sem_w0, sem_w1)

        def fire_gathers(ci, p):
            cps = []
            for j in range(JPC):
                row = ci * JPC + j
                cps.append(pltpu.async_copy(
                    t_hbm.at[il_v.at[row]],
                    rl_v.at[p].at[pl.ds(j * 128, 128)], sem_l))
                cps.append(pltpu.async_copy(
                    t_hbm.at[ir_v.at[row]],
                    rr_v.at[p].at[pl.ds(j * 128, 128)], sem_r))
            return cps

        def fire_writeback(ci, p):
            base = wid * rows_per_w + ci * CHUNK
            return [
                pltpu.async_copy(
                    rl_v.at[p], o_hbm.at[pl.ds(base, CHUNK), pl.ds(0, D)],
                    wsems[p]),
                pltpu.async_copy(
                    rr_v.at[p], o_hbm.at[pl.ds(base, CHUNK), pl.ds(D, D)],
                    wsems[p]),
            ]

        pending_wb = [None, None]
        gathers = fire_gathers(0, 0)
        for ci in range(n_chunk):
            p = ci % 2
            for cp in gathers:
                cp.wait()
            if ci + 1 < n_chunk:
                q = 1 - p
                if pending_wb[q] is not None:
                    for cp in pending_wb[q]:
                        cp.wait()
                gathers = fire_gathers(ci + 1, q)
            pending_wb[p] = fire_writeback(ci, p)
        for p in (0, 1):
            if pending_wb[p] is not None:
                for cp in pending_wb[p]:
                    cp.wait()

    return gather_kernel(table, il, ir)


def _mlp(f, w1, b1, w2, b2):
    n_rows, D2 = f.shape
    H = w1.shape[1]
    O = w2.shape[0]
    TM = 16384
    grid = (n_rows // TM,)

    def body(f_ref, w1_ref, b1_ref, w2t_ref, b2_ref, out_ref):
        h = jnp.dot(f_ref[...], w1_ref[...],
                    preferred_element_type=jnp.float32)
        h = jnp.maximum(h + b1_ref[...], 0.0)
        pt = jax.lax.dot_general(
            w2t_ref[...], h, (((1,), (1,)), ((), ())),
            preferred_element_type=jnp.float32)
        out_ref[...] = pt + b2_ref[...]

    return pl.pallas_call(
        body,
        grid=grid,
        in_specs=[
            pl.BlockSpec((TM, D2), lambda i: (i, 0)),
            pl.BlockSpec((D2, H), lambda i: (0, 0)),
            pl.BlockSpec((1, H), lambda i: (0, 0)),
            pl.BlockSpec((O, H), lambda i: (0, 0)),
            pl.BlockSpec((O, 1), lambda i: (0, 0)),
        ],
        out_specs=pl.BlockSpec((O, TM), lambda i: (0, i)),
        out_shape=jax.ShapeDtypeStruct((O, n_rows), jnp.float32),
    )(f, w1, b1, w2, b2)


def kernel(soutput_f_l, soutput_f_r, matches, non_matches, start_idx,
           num_points, W1, b1, W2, b2):
    B, M, _ = matches.shape
    NM = non_matches.shape[1]
    R, D = soutput_f_l.shape
    n_static = R // B

    def global_idx(x):
        x = jnp.where(x > -1, x, num_points[:, None])
        x = jnp.clip(x, 0, n_static - 1)
        return x + start_idx[:, None]

    il = 2 * jnp.concatenate(
        [global_idx(matches[:, :, 0]), global_idx(non_matches[:, :, 0])],
        axis=0)
    ir = 2 * jnp.concatenate(
        [global_idx(matches[:, :, 1]), global_idx(non_matches[:, :, 1])],
        axis=0) + 1
    n_rows = B * (M + NM)
    il = il.reshape(n_rows // 128, 128)
    ir = ir.reshape(n_rows // 128, 128)

    table = jnp.concatenate([soutput_f_l, soutput_f_r],
                            axis=1).reshape(2 * R, D)

    NSPLIT = 4
    idx_rows = n_rows // 128
    srows = idx_rows // NSPLIT
    preds = []
    for s in range(NSPLIT):
        il_s = jax.lax.slice_in_dim(il, s * srows, (s + 1) * srows, axis=0)
        ir_s = jax.lax.slice_in_dim(ir, s * srows, (s + 1) * srows, axis=0)
        f_s = _fused_gather(table, il_s, ir_s)
        preds.append(_mlp(f_s, W1, b1.reshape(1, -1), W2.T,
                          b2.reshape(-1, 1)))
    prediction = jnp.concatenate(preds, axis=1).T
    label = jnp.concatenate(
        [jnp.ones((B * M,), jnp.float32), jnp.zeros((B * NM,), jnp.float32)])
    return (prediction, label)


# speedup vs baseline: 1.1062x; 1.1062x over previous
import functools

import jax
import jax.numpy as jnp
from jax import lax
from jax.experimental import pallas as pl
from jax.experimental.pallas import tpu as pltpu
from jax.experimental.pallas import tpu_sc as plsc


def _fused_gather(table, il, ir):
    n_rows = il.shape[0] * 128
    D = table.shape[1]
    info = plsc.get_sparse_core_info()
    NC, NS = info.num_cores, info.num_subcores
    NW = NC * NS
    rows_per_w = n_rows // NW
    CHUNK = 256
    n_chunk = rows_per_w // CHUNK
    JPC = CHUNK // 128
    idx_rows_w = rows_per_w // 128

    mesh = plsc.VectorSubcoreMesh(core_axis_name="c", subcore_axis_name="s")

    @functools.partial(
        pl.kernel,
        mesh=mesh,
        compiler_params=pltpu.CompilerParams(use_tc_tiling_on_sc=False),
        out_type=jax.ShapeDtypeStruct((n_rows, 2 * D), jnp.float32),
        scratch_types=[
            pltpu.VMEM((idx_rows_w, 128), jnp.int32),
            pltpu.VMEM((idx_rows_w, 128), jnp.int32),
            pltpu.VMEM((2, CHUNK, D), jnp.float32),
            pltpu.VMEM((2, CHUNK, D), jnp.float32),
            pltpu.SemaphoreType.DMA,
            pltpu.SemaphoreType.DMA,
            pltpu.SemaphoreType.DMA,
            pltpu.SemaphoreType.DMA,
        ],
    )
    def gather_kernel(t_hbm, il_hbm, ir_hbm, o_hbm,
                      il_v, ir_v, rl_v, rr_v, sem_l, sem_r, sem_w0, sem_w1):
        wid = lax.axis_index("s") * NC + lax.axis_index("c")
        pltpu.sync_copy(il_hbm.at[pl.ds(wid * idx_rows_w, idx_rows_w)], il_v)
        pltpu.sync_copy(ir_hbm.at[pl.ds(wid * idx_rows_w, idx_rows_w)], ir_v)
        wsems = (sem_w0, sem_w1)

        def fire_gathers(ci, p):
            cps = []
            for j in range(JPC):
                row = ci * JPC + j
                cps.append(pltpu.async_copy(
                    t_hbm.at[il_v.at[row]],
                    rl_v.at[p].at[pl.ds(j * 128, 128)], sem_l))
                cps.append(pltpu.async_copy(
                    t_hbm.at[ir_v.at[row]],
                    rr_v.at[p].at[pl.ds(j * 128, 128)], sem_r))
            return cps

        def fire_writeback(ci, p):
            base = wid * rows_per_w + ci * CHUNK
            return [
                pltpu.async_copy(
                    rl_v.at[p], o_hbm.at[pl.ds(base, CHUNK), pl.ds(0, D)],
                    wsems[p]),
                pltpu.async_copy(
                    rr_v.at[p], o_hbm.at[pl.ds(base, CHUNK), pl.ds(D, D)],
                    wsems[p]),
            ]

        pending_wb = [None, None]
        gathers = fire_gathers(0, 0)
        for ci in range(n_chunk):
            p = ci % 2
            for cp in gathers:
                cp.wait()
            if ci + 1 < n_chunk:
                q = 1 - p
                if pending_wb[q] is not None:
                    for cp in pending_wb[q]:
                        cp.wait()
                gathers = fire_gathers(ci + 1, q)
            pending_wb[p] = fire_writeback(ci, p)
        for p in (0, 1):
            if pending_wb[p] is not None:
                for cp in pending_wb[p]:
                    cp.wait()

    return gather_kernel(table, il, ir)


def _mlp(f, w1, b1, w2, b2):
    n_rows, D2 = f.shape
    H = w1.shape[1]
    O = w2.shape[0]
    TM = 16384
    grid = (n_rows // TM,)

    def body(f_ref, w1_ref, b1_ref, w2t_ref, b2_ref, out_ref):
        h = jnp.dot(f_ref[...], w1_ref[...],
                    preferred_element_type=jnp.float32)
        h = jnp.maximum(h + b1_ref[...], 0.0)
        pt = jax.lax.dot_general(
            w2t_ref[...], h, (((1,), (1,)), ((), ())),
            preferred_element_type=jnp.float32)
        out_ref[...] = pt + b2_ref[...]

    return pl.pallas_call(
        body,
        grid=grid,
        in_specs=[
            pl.BlockSpec((TM, D2), lambda i: (i, 0)),
            pl.BlockSpec((D2, H), lambda i: (0, 0)),
            pl.BlockSpec((1, H), lambda i: (0, 0)),
            pl.BlockSpec((O, H), lambda i: (0, 0)),
            pl.BlockSpec((O, 1), lambda i: (0, 0)),
        ],
        out_specs=pl.BlockSpec((O, TM), lambda i: (0, i)),
        out_shape=jax.ShapeDtypeStruct((O, n_rows), jnp.float32),
    )(f, w1, b1, w2, b2)


def kernel(soutput_f_l, soutput_f_r, matches, non_matches, start_idx,
           num_points, W1, b1, W2, b2):
    B, M, _ = matches.shape
    NM = non_matches.shape[1]
    R, D = soutput_f_l.shape
    n_static = R // B

    def global_idx(x):
        x = jnp.where(x > -1, x, num_points[:, None])
        x = jnp.clip(x, 0, n_static - 1)
        return x + start_idx[:, None]

    il = 2 * jnp.concatenate(
        [global_idx(matches[:, :, 0]), global_idx(non_matches[:, :, 0])],
        axis=0)
    ir = 2 * jnp.concatenate(
        [global_idx(matches[:, :, 1]), global_idx(non_matches[:, :, 1])],
        axis=0) + 1
    n_rows = B * (M + NM)
    il = il.reshape(n_rows // 128, 128)
    ir = ir.reshape(n_rows // 128, 128)

    table = jnp.concatenate([soutput_f_l, soutput_f_r],
                            axis=1).reshape(2 * R, D)

    NSPLIT = 2
    idx_rows = n_rows // 128
    srows = idx_rows // NSPLIT
    preds = []
    for s in range(NSPLIT):
        il_s = jax.lax.slice_in_dim(il, s * srows, (s + 1) * srows, axis=0)
        ir_s = jax.lax.slice_in_dim(ir, s * srows, (s + 1) * srows, axis=0)
        f_s = _fused_gather(table, il_s, ir_s)
        preds.append(_mlp(f_s, W1, b1.reshape(1, -1), W2.T,
                          b2.reshape(-1, 1)))
    prediction = jnp.concatenate(preds, axis=1).T
    label = jnp.concatenate(
        [jnp.ones((B * M,), jnp.float32), jnp.zeros((B * NM,), jnp.float32)])
    return (prediction, label)


# speedup vs baseline: 1.1260x; 1.0179x over previous
import functools

import jax
import jax.numpy as jnp
from jax import lax
from jax.experimental import pallas as pl
from jax.experimental.pallas import tpu as pltpu
from jax.experimental.pallas import tpu_sc as plsc


def _fused_gather(table, il, ir):
    n_rows = il.shape[0] * 128
    D = table.shape[1]
    info = plsc.get_sparse_core_info()
    NC, NS = info.num_cores, info.num_subcores
    NW = NC * NS
    rows_per_w = n_rows // NW
    CHUNK = 256
    n_chunk = rows_per_w // CHUNK
    JPC = CHUNK // 128
    idx_rows_w = rows_per_w // 128

    mesh = plsc.VectorSubcoreMesh(core_axis_name="c", subcore_axis_name="s")

    @functools.partial(
        pl.kernel,
        mesh=mesh,
        compiler_params=pltpu.CompilerParams(use_tc_tiling_on_sc=False),
        out_type=jax.ShapeDtypeStruct((n_rows, 2 * D), jnp.float32),
        scratch_types=[
            pltpu.VMEM((idx_rows_w, 128), jnp.int32),
            pltpu.VMEM((idx_rows_w, 128), jnp.int32),
            pltpu.VMEM((3, CHUNK, D), jnp.float32),
            pltpu.VMEM((3, CHUNK, D), jnp.float32),
            pltpu.SemaphoreType.DMA,
            pltpu.SemaphoreType.DMA,
            pltpu.SemaphoreType.DMA,
            pltpu.SemaphoreType.DMA,
            pltpu.SemaphoreType.DMA,
        ],
    )
    def gather_kernel(t_hbm, il_hbm, ir_hbm, o_hbm,
                      il_v, ir_v, rl_v, rr_v, sem_l, sem_r, sem_w0, sem_w1, sem_w2):
        wid = lax.axis_index("s") * NC + lax.axis_index("c")
        pltpu.sync_copy(il_hbm.at[pl.ds(wid * idx_rows_w, idx_rows_w)], il_v)
        pltpu.sync_copy(ir_hbm.at[pl.ds(wid * idx_rows_w, idx_rows_w)], ir_v)
        wsems = (sem_w0, sem_w1, sem_w2)

        def fire_gathers(ci, p):
            cps = []
            for j in range(JPC):
                row = ci * JPC + j
                cps.append(pltpu.async_copy(
                    t_hbm.at[il_v.at[row]],
                    rl_v.at[p].at[pl.ds(j * 128, 128)], sem_l))
                cps.append(pltpu.async_copy(
                    t_hbm.at[ir_v.at[row]],
                    rr_v.at[p].at[pl.ds(j * 128, 128)], sem_r))
            return cps

        def fire_writeback(ci, p):
            base = wid * rows_per_w + ci * CHUNK
            return [
                pltpu.async_copy(
                    rl_v.at[p], o_hbm.at[pl.ds(base, CHUNK), pl.ds(0, D)],
                    wsems[p]),
                pltpu.async_copy(
                    rr_v.at[p], o_hbm.at[pl.ds(base, CHUNK), pl.ds(D, D)],
                    wsems[p]),
            ]

        NB = 3
        pending_wb = [None] * NB
        pending_g = [None] * NB
        pending_g[0] = fire_gathers(0, 0)
        if n_chunk > 1:
            pending_g[1] = fire_gathers(1, 1)
        for ci in range(n_chunk):
            p = ci % NB
            for cp in pending_g[p]:
                cp.wait()
            nxt = ci + 2
            if nxt < n_chunk:
                q = nxt % NB
                if pending_wb[q] is not None:
                    for cp in pending_wb[q]:
                        cp.wait()
                    pending_wb[q] = None
                pending_g[q] = fire_gathers(nxt, q)
            pending_wb[p] = fire_writeback(ci, p)
        for p in range(NB):
            if pending_wb[p] is not None:
                for cp in pending_wb[p]:
                    cp.wait()

    return gather_kernel(table, il, ir)


def _mlp(f, w1, b1, w2, b2):
    n_rows, D2 = f.shape
    H = w1.shape[1]
    O = w2.shape[0]
    TM = 16384
    grid = (n_rows // TM,)

    def body(f_ref, w1_ref, b1_ref, w2t_ref, b2_ref, out_ref):
        h = jnp.dot(f_ref[...], w1_ref[...],
                    preferred_element_type=jnp.float32)
        h = jnp.maximum(h + b1_ref[...], 0.0)
        pt = jax.lax.dot_general(
            w2t_ref[...], h, (((1,), (1,)), ((), ())),
            preferred_element_type=jnp.float32)
        out_ref[...] = pt + b2_ref[...]

    return pl.pallas_call(
        body,
        grid=grid,
        in_specs=[
            pl.BlockSpec((TM, D2), lambda i: (i, 0)),
            pl.BlockSpec((D2, H), lambda i: (0, 0)),
            pl.BlockSpec((1, H), lambda i: (0, 0)),
            pl.BlockSpec((O, H), lambda i: (0, 0)),
            pl.BlockSpec((O, 1), lambda i: (0, 0)),
        ],
        out_specs=pl.BlockSpec((O, TM), lambda i: (0, i)),
        out_shape=jax.ShapeDtypeStruct((O, n_rows), jnp.float32),
    )(f, w1, b1, w2, b2)


def kernel(soutput_f_l, soutput_f_r, matches, non_matches, start_idx,
           num_points, W1, b1, W2, b2):
    B, M, _ = matches.shape
    NM = non_matches.shape[1]
    R, D = soutput_f_l.shape
    n_static = R // B

    def global_idx(x):
        x = jnp.where(x > -1, x, num_points[:, None])
        x = jnp.clip(x, 0, n_static - 1)
        return x + start_idx[:, None]

    il = 2 * jnp.concatenate(
        [global_idx(matches[:, :, 0]), global_idx(non_matches[:, :, 0])],
        axis=0)
    ir = 2 * jnp.concatenate(
        [global_idx(matches[:, :, 1]), global_idx(non_matches[:, :, 1])],
        axis=0) + 1
    n_rows = B * (M + NM)
    il = il.reshape(n_rows // 128, 128)
    ir = ir.reshape(n_rows // 128, 128)

    table = jnp.concatenate([soutput_f_l, soutput_f_r],
                            axis=1).reshape(2 * R, D)

    NSPLIT = 2
    idx_rows = n_rows // 128
    srows = idx_rows // NSPLIT
    preds = []
    for s in range(NSPLIT):
        il_s = jax.lax.slice_in_dim(il, s * srows, (s + 1) * srows, axis=0)
        ir_s = jax.lax.slice_in_dim(ir, s * srows, (s + 1) * srows, axis=0)
        f_s = _fused_gather(table, il_s, ir_s)
        preds.append(_mlp(f_s, W1, b1.reshape(1, -1), W2.T,
                          b2.reshape(-1, 1)))
    prediction = jnp.concatenate(preds, axis=1).T
    label = jnp.concatenate(
        [jnp.ones((B * M,), jnp.float32), jnp.zeros((B * NM,), jnp.float32)])
    return (prediction, label)


# speedup vs baseline: 1.1262x; 1.0002x over previous
import functools

import jax
import jax.numpy as jnp
from jax import lax
from jax.experimental import pallas as pl
from jax.experimental.pallas import tpu as pltpu
from jax.experimental.pallas import tpu_sc as plsc


def _fused_gather(table, il, ir, base_idx_row, n_slice_rows):
    n_rows = n_slice_rows
    D = table.shape[1]
    info = plsc.get_sparse_core_info()
    NC, NS = info.num_cores, info.num_subcores
    NW = NC * NS
    rows_per_w = n_rows // NW
    CHUNK = 256
    n_chunk = rows_per_w // CHUNK
    JPC = CHUNK // 128
    idx_rows_w = rows_per_w // 128

    mesh = plsc.VectorSubcoreMesh(core_axis_name="c", subcore_axis_name="s")

    @functools.partial(
        pl.kernel,
        mesh=mesh,
        compiler_params=pltpu.CompilerParams(use_tc_tiling_on_sc=False),
        out_type=jax.ShapeDtypeStruct((n_rows, 2 * D), jnp.float32),
        scratch_types=[
            pltpu.VMEM((idx_rows_w, 128), jnp.int32),
            pltpu.VMEM((idx_rows_w, 128), jnp.int32),
            pltpu.VMEM((3, CHUNK, D), jnp.float32),
            pltpu.VMEM((3, CHUNK, D), jnp.float32),
            pltpu.SemaphoreType.DMA,
            pltpu.SemaphoreType.DMA,
            pltpu.SemaphoreType.DMA,
            pltpu.SemaphoreType.DMA,
            pltpu.SemaphoreType.DMA,
        ],
    )
    def gather_kernel(t_hbm, il_hbm, ir_hbm, o_hbm,
                      il_v, ir_v, rl_v, rr_v, sem_l, sem_r, sem_w0, sem_w1, sem_w2):
        wid = lax.axis_index("s") * NC + lax.axis_index("c")
        pltpu.sync_copy(
            il_hbm.at[pl.ds(base_idx_row + wid * idx_rows_w, idx_rows_w)],
            il_v)
        pltpu.sync_copy(
            ir_hbm.at[pl.ds(base_idx_row + wid * idx_rows_w, idx_rows_w)],
            ir_v)
        wsems = (sem_w0, sem_w1, sem_w2)

        def fire_gathers(ci, p):
            cps = []
            for j in range(JPC):
                row = ci * JPC + j
                cps.append(pltpu.async_copy(
                    t_hbm.at[il_v.at[row]],
                    rl_v.at[p].at[pl.ds(j * 128, 128)], sem_l))
                cps.append(pltpu.async_copy(
                    t_hbm.at[ir_v.at[row]],
                    rr_v.at[p].at[pl.ds(j * 128, 128)], sem_r))
            return cps

        def fire_writeback(ci, p):
            base = wid * rows_per_w + ci * CHUNK
            return [
                pltpu.async_copy(
                    rl_v.at[p], o_hbm.at[pl.ds(base, CHUNK), pl.ds(0, D)],
                    wsems[p]),
                pltpu.async_copy(
                    rr_v.at[p], o_hbm.at[pl.ds(base, CHUNK), pl.ds(D, D)],
                    wsems[p]),
            ]

        NB = 3
        pending_wb = [None] * NB
        pending_g = [None] * NB
        pending_g[0] = fire_gathers(0, 0)
        if n_chunk > 1:
            pending_g[1] = fire_gathers(1, 1)
        for ci in range(n_chunk):
            p = ci % NB
            for cp in pending_g[p]:
                cp.wait()
            nxt = ci + 2
            if nxt < n_chunk:
                q = nxt % NB
                if pending_wb[q] is not None:
                    for cp in pending_wb[q]:
                        cp.wait()
                    pending_wb[q] = None
                pending_g[q] = fire_gathers(nxt, q)
            pending_wb[p] = fire_writeback(ci, p)
        for p in range(NB):
            if pending_wb[p] is not None:
                for cp in pending_wb[p]:
                    cp.wait()

    return gather_kernel(table, il, ir)


def _mlp(f, w1, b1, w2, b2):
    n_rows, D2 = f.shape
    H = w1.shape[1]
    O = w2.shape[0]
    TM = 16384
    grid = (n_rows // TM,)

    def body(f_ref, w1_ref, b1_ref, w2t_ref, b2_ref, out_ref):
        h = jnp.dot(f_ref[...], w1_ref[...],
                    preferred_element_type=jnp.float32)
        h = jnp.maximum(h + b1_ref[...], 0.0)
        pt = jax.lax.dot_general(
            w2t_ref[...], h, (((1,), (1,)), ((), ())),
            preferred_element_type=jnp.float32)
        out_ref[...] = pt + b2_ref[...]

    return pl.pallas_call(
        body,
        grid=grid,
        in_specs=[
            pl.BlockSpec((TM, D2), lambda i: (i, 0)),
            pl.BlockSpec((D2, H), lambda i: (0, 0)),
            pl.BlockSpec((1, H), lambda i: (0, 0)),
            pl.BlockSpec((O, H), lambda i: (0, 0)),
            pl.BlockSpec((O, 1), lambda i: (0, 0)),
        ],
        out_specs=pl.BlockSpec((O, TM), lambda i: (0, i)),
        out_shape=jax.ShapeDtypeStruct((O, n_rows), jnp.float32),
    )(f, w1, b1, w2, b2)


def kernel(soutput_f_l, soutput_f_r, matches, non_matches, start_idx,
           num_points, W1, b1, W2, b2):
    B, M, _ = matches.shape
    NM = non_matches.shape[1]
    R, D = soutput_f_l.shape
    n_static = R // B

    def global_idx(x):
        x = jnp.where(x > -1, x, num_points[:, None])
        x = jnp.clip(x, 0, n_static - 1)
        return x + start_idx[:, None]

    il = 2 * jnp.concatenate(
        [global_idx(matches[:, :, 0]), global_idx(non_matches[:, :, 0])],
        axis=0)
    ir = 2 * jnp.concatenate(
        [global_idx(matches[:, :, 1]), global_idx(non_matches[:, :, 1])],
        axis=0) + 1
    n_rows = B * (M + NM)
    il = il.reshape(n_rows // 128, 128)
    ir = ir.reshape(n_rows // 128, 128)

    table = jnp.concatenate([soutput_f_l, soutput_f_r],
                            axis=1).reshape(2 * R, D)

    NSPLIT = 2
    idx_rows = n_rows // 128
    srows = idx_rows // NSPLIT
    preds = []
    for s in range(NSPLIT):
        f_s = _fused_gather(table, il, ir, s * srows, srows * 128)
        preds.append(_mlp(f_s, W1, b1.reshape(1, -1), W2.T,
                          b2.reshape(-1, 1)))
    prediction = jnp.concatenate(preds, axis=1).T
    label = jnp.concatenate(
        [jnp.ones((B * M,), jnp.float32), jnp.zeros((B * NM,), jnp.float32)])
    return (prediction, label)
